# baseline (speedup 1.0000x reference)
import functools

import jax
import jax.numpy as jnp
from jax import lax
from jax.experimental import pallas as pl
from jax.experimental.pallas import tpu as pltpu
from jax.experimental.pallas import tpu_sc as plsc

_B, _S, _D = 1024, 200, 64
_V = 1000000
_N = _B * _S
_NC, _NS = 2, 16
_NW = _NC * _NS
_CH = 128
_CPW = _N // (_NW * _CH)
_K = 5
assert _CPW % _K == 0 and _N % (_CH * _NW) == 0

_mesh = plsc.VectorSubcoreMesh(core_axis_name="c", subcore_axis_name="s")


@functools.partial(
    pl.kernel,
    out_type=jax.ShapeDtypeStruct((_N, 2 * _D), jnp.uint16),
    mesh=_mesh,
    compiler_params=pltpu.CompilerParams(
        needs_layout_passes=False, use_tc_tiling_on_sc=False
    ),
    scratch_types=[
        pltpu.VMEM((_CPW, _CH), jnp.int32),
        pltpu.VMEM((_K, _CH, 2 * _D), jnp.uint16),
        pltpu.SemaphoreType.DMA,
        pltpu.SemaphoreType.DMA,
    ],
)
def _sc_gather(idx_hbm, t16_hbm, out_hbm, idx_v, rows_v, gsem, osem):
    wid = lax.axis_index("s") * _NC + lax.axis_index("c")
    base = wid * _CPW
    pltpu.sync_copy(idx_hbm.at[wid], idx_v)

    @pl.loop(0, _CPW, step=_K)
    def _step(j):
        gathers = [
            pltpu.async_copy(t16_hbm.at[idx_v.at[j + b]], rows_v.at[b], gsem)
            for b in range(_K)
        ]
        stores = []
        for b in range(_K):
            gathers[b].wait()
            stores.append(
                pltpu.async_copy(
                    rows_v.at[b],
                    out_hbm.at[pl.ds((base + j + b) * _CH, _CH)],
                    osem,
                )
            )
        for st in stores:
            st.wait()


def kernel(inp_ids, table):
    idx = inp_ids.reshape(_NW, _CPW, _CH)
    t16 = lax.bitcast_convert_type(table, jnp.uint16).reshape(_V, 2 * _D)
    out16 = _sc_gather(idx, t16)
    out = lax.bitcast_convert_type(out16.reshape(_N, _D, 2), jnp.float32)
    return out.reshape(_B, _S, _D)


# speedup vs baseline: 3.8288x; 3.8288x over previous
import functools

import jax
import jax.numpy as jnp
from jax import lax
from jax.experimental import pallas as pl
from jax.experimental.pallas import tpu as pltpu
from jax.experimental.pallas import tpu_sc as plsc

_B, _S, _D = 1024, 200, 64
_V = 1000000
_N = _B * _S
_NC, _NS = 2, 16
_NW = _NC * _NS
_CH = 128
_CPW = _N // (_NW * _CH)
_K = 5
assert _CPW % _K == 0 and _N % (_CH * _NW) == 0

_mesh = plsc.VectorSubcoreMesh(core_axis_name="c", subcore_axis_name="s")


@functools.partial(
    pl.kernel,
    out_type=jax.ShapeDtypeStruct((_N, 2 * _D), jnp.float32),
    mesh=_mesh,
    compiler_params=pltpu.CompilerParams(
        needs_layout_passes=False, use_tc_tiling_on_sc=False
    ),
    scratch_types=[
        pltpu.VMEM((_CPW, _CH), jnp.int32),
        pltpu.VMEM((_K, _CH, 2 * _D), jnp.float32),
        pltpu.SemaphoreType.DMA,
        pltpu.SemaphoreType.DMA,
    ],
)
def _sc_gather(idx_hbm, t2_hbm, out_hbm, idx_v, rows_v, gsem, osem):
    wid = lax.axis_index("s") * _NC + lax.axis_index("c")
    base = wid * _CPW
    pltpu.sync_copy(idx_hbm.at[wid], idx_v)

    @pl.loop(0, _CPW, step=_K)
    def _step(j):
        gathers = [
            pltpu.async_copy(t2_hbm.at[idx_v.at[j + b]], rows_v.at[b], gsem)
            for b in range(_K)
        ]
        stores = []
        for b in range(_K):
            gathers[b].wait()
            stores.append(
                pltpu.async_copy(
                    rows_v.at[b],
                    out_hbm.at[pl.ds((base + j + b) * _CH, _CH)],
                    osem,
                )
            )
        for st in stores:
            st.wait()


def kernel(inp_ids, table):
    idxt = inp_ids.T
    ev = lax.shift_right_logical(idxt, 1).reshape(_NW, _CPW, _CH)
    t2 = table.reshape(_V // 2, 2 * _D)
    pairs = _sc_gather(ev, t2)
    pairs = pairs.reshape(_S, _B, 2, _D)
    odd = lax.bitwise_and(idxt, 1)[:, :, None].astype(jnp.bool_)
    out = jnp.where(odd, pairs[:, :, 1, :], pairs[:, :, 0, :])
    return out.transpose(1, 0, 2)


# speedup vs baseline: 4.8949x; 1.2784x over previous
import functools

import jax
import jax.numpy as jnp
from jax import lax
from jax.experimental import pallas as pl
from jax.experimental.pallas import tpu as pltpu
from jax.experimental.pallas import tpu_sc as plsc

_B, _S, _D = 1024, 200, 64
_V = 1000000
_N = _B * _S
_NC, _NS = 2, 16
_NW = _NC * _NS
_CH = 128
_CPW = _N // (_NW * _CH)
_K = 5
assert _CPW % _K == 0 and _N % (_CH * _NW) == 0

_mesh = plsc.VectorSubcoreMesh(core_axis_name="c", subcore_axis_name="s")


@functools.partial(
    pl.kernel,
    out_type=jax.ShapeDtypeStruct((_N, _D), jnp.float32),
    mesh=_mesh,
    compiler_params=pltpu.CompilerParams(
        needs_layout_passes=False, use_tc_tiling_on_sc=False
    ),
    scratch_types=[
        pltpu.VMEM((_CPW, _CH), jnp.int32),
        pltpu.VMEM((_K, _CH, 2 * _D), jnp.float32),
        pltpu.SemaphoreType.DMA,
        pltpu.SemaphoreType.DMA,
    ],
)
def _sc_gather(idx_hbm, table_hbm, out_hbm, idx_v, rows_v, gsem, osem):
    wid = lax.axis_index("s") * _NC + lax.axis_index("c")
    base = wid * _CPW
    pltpu.sync_copy(idx_hbm.at[wid], idx_v)

    @pl.loop(0, _CPW, step=_K)
    def _step(j):
        gathers = [
            pltpu.async_copy(table_hbm.at[idx_v.at[j + b]], rows_v.at[b], gsem)
            for b in range(_K)
        ]
        stores = []
        for b in range(_K):
            gathers[b].wait()
            stores.append(
                pltpu.async_copy(
                    rows_v.at[b].at[:, pl.ds(0, _D)],
                    out_hbm.at[pl.ds((base + j + b) * _CH, _CH)],
                    osem,
                )
            )
        for st in stores:
            st.wait()


def kernel(inp_ids, table):
    idx = inp_ids.T.reshape(_NW, _CPW, _CH)
    tpad = jnp.pad(table, ((0, 0), (0, 128 - _D)))
    out = _sc_gather(idx, tpad)
    return out.reshape(_S, _B, _D).transpose(1, 0, 2)
